# baseline (speedup 1.0000x reference)
import functools

import jax
import jax.numpy as jnp
from jax import lax
from jax.experimental import pallas as pl
from jax.experimental.pallas import tpu as pltpu
from jax.experimental.pallas import tpu_sc as plsc

N = 10000
D = 128
E = 320000
G = 64

NC = 2
NS = 16

CHUNK = 128
CH_TOT = -(-E // CHUNK)
CPT = ((CH_TOT + NS - 1) // NS + 7) // 8 * 8
CH_TOT_PAD = CPT * NS
E_PAD = CH_TOT_PAD * CHUNK

N_PAD = 10240
ROWS_PER_TILE = N_PAD // NS

NBUF = 2
SS = 16
FAST_CORE = 0
CPT_FAST = 128
CPT_SLOW = 32


def _agg_body(x_hbm, src_hbm, dst_hbm, out_hbm, src_idx, dst_idx, bufs, acc,
              sems):
    cid = lax.axis_index("c")
    tid = lax.axis_index("s")

    rows = bufs[0]

    def zero_body(t, _):
        i = t // (D // 16)
        k = t % (D // 16)
        rows[i, pl.ds(k * 16, 16)] = jnp.zeros((16,), jnp.float32)
        return 0
    lax.fori_loop(0, CHUNK * (D // 16), zero_body, 0)
    base = tid * ROWS_PER_TILE
    for c in range(ROWS_PER_TILE // CHUNK):
        pltpu.sync_copy(rows, acc.at[pl.ds(base + c * CHUNK, CHUNK)])
    plsc.subcore_barrier()

    def pipeline(chunk_base, cpt):
        for st in range(cpt // SS):
            off = chunk_base + st * SS
            pltpu.sync_copy(src_hbm.at[pl.ds(off, SS)], src_idx)
            pltpu.sync_copy(dst_hbm.at[pl.ds(off, SS)], dst_idx)

            for b in range(NBUF):
                pltpu.async_copy(x_hbm.at[src_idx.at[b]], bufs[b], sems[b])

            def group_body(g, _):
                for b in range(NBUF):
                    j = g * NBUF + b
                    pltpu.make_async_copy(x_hbm.at[src_idx.at[0]], bufs[b],
                                          sems[b]).wait()
                    pltpu.sync_copy(bufs[b], acc.at[dst_idx.at[j]], add=True)
                    nj = j + NBUF

                    @pl.when(nj < SS)
                    def _():
                        pltpu.async_copy(x_hbm.at[src_idx.at[nj]], bufs[b],
                                         sems[b])
                return 0
            lax.fori_loop(0, SS // NBUF, group_body, 0)

    @pl.when(cid == FAST_CORE)
    def _():
        pipeline(tid * CPT_FAST, CPT_FAST)

    @pl.when(cid == 1 - FAST_CORE)
    def _():
        pipeline(NS * CPT_FAST + tid * CPT_SLOW, CPT_SLOW)

    plsc.subcore_barrier()

    pltpu.sync_copy(acc.at[pl.ds(base, ROWS_PER_TILE)],
                    out_hbm.at[cid].at[pl.ds(base, ROWS_PER_TILE)])


@functools.partial(
    pl.kernel,
    out_type=jax.ShapeDtypeStruct((NC, N_PAD, D), jnp.float32),
    mesh=plsc.VectorSubcoreMesh(core_axis_name="c", subcore_axis_name="s"),
    scratch_types=[
        pltpu.VMEM((SS, CHUNK), jnp.int32),
        pltpu.VMEM((SS, CHUNK), jnp.int32),
        pltpu.VMEM((CHUNK, D), jnp.float32),
        pltpu.VMEM((CHUNK, D), jnp.float32),
        pltpu.VMEM_SHARED((N_PAD, D), jnp.float32),
        pltpu.SemaphoreType.DMA,
        pltpu.SemaphoreType.DMA,
    ],
)
def _sc_aggregate(x_hbm, src_hbm, dst_hbm, out_hbm, src_idx, dst_idx,
                  b0, b1, acc, s0, s1):
    _agg_body(x_hbm, src_hbm, dst_hbm, out_hbm, src_idx, dst_idx,
              [b0, b1], acc, [s0, s1])


BN = 2000
GRID = N // BN


def _mlp_body(do_pool, x_ref, a0_ref, a1_ref, w1_ref, b1_ref, w2_ref, b2_ref,
              *rest):
    if do_pool:
        (batch_ref, lw_ref, lb_ref, h_ref, out_ref, pooled) = rest
    else:
        (h_ref,) = rest
    h = x_ref[...] + a0_ref[...] + a1_ref[...]
    h = jnp.maximum(
        lax.dot_general(h, w1_ref[...], (((1,), (0,)), ((), ())),
                        preferred_element_type=jnp.float32) + b1_ref[...], 0.0)
    h = lax.dot_general(h, w2_ref[...], (((1,), (0,)), ((), ())),
                        preferred_element_type=jnp.float32) + b2_ref[...]
    h = jnp.maximum(h, 0.0)
    h_ref[...] = h
    if do_pool:
        i = pl.program_id(0)

        @pl.when(i == 0)
        def _():
            pooled[...] = jnp.zeros((G, D), jnp.float32)

        seg = batch_ref[0]
        oh = (lax.broadcasted_iota(jnp.int32, (G, BN), 0) == seg
              ).astype(jnp.float32)
        pooled[...] += lax.dot_general(oh, h, (((1,), (0,)), ((), ())),
                                       preferred_element_type=jnp.float32)

        @pl.when(i == GRID - 1)
        def _():
            z = lax.dot_general(pooled[...], lw_ref[...],
                                (((1,), (0,)), ((), ())),
                                preferred_element_type=jnp.float32)
            z = z + lb_ref[0, 0]
            out_ref[...] = 1.0 / (1.0 + jnp.exp(-z))


def _make_mlp(do_pool):
    in_specs = [
        pl.BlockSpec((BN, D), lambda i: (i, 0)),
        pl.BlockSpec((BN, D), lambda i: (i, 0)),
        pl.BlockSpec((BN, D), lambda i: (i, 0)),
        pl.BlockSpec((D, D), lambda i: (0, 0)),
        pl.BlockSpec((1, D), lambda i: (0, 0)),
        pl.BlockSpec((D, D), lambda i: (0, 0)),
        pl.BlockSpec((1, D), lambda i: (0, 0)),
    ]
    out_specs = pl.BlockSpec((BN, D), lambda i: (i, 0))
    out_shape = jax.ShapeDtypeStruct((N, D), jnp.float32)
    scratch = []
    if do_pool:
        in_specs += [
            pl.BlockSpec((1, 1, BN), lambda i: (i, 0, 0)),
            pl.BlockSpec((D, D), lambda i: (0, 0)),
            pl.BlockSpec((1, D), lambda i: (0, 0)),
        ]
        out_specs = [out_specs, pl.BlockSpec((G, D), lambda i: (0, 0))]
        out_shape = [out_shape, jax.ShapeDtypeStruct((G, D), jnp.float32)]
        scratch = [pltpu.VMEM((G, D), jnp.float32)]
    return pl.pallas_call(
        functools.partial(_mlp_body, do_pool),
        grid=(GRID,),
        in_specs=in_specs,
        out_specs=out_specs,
        out_shape=out_shape,
        scratch_shapes=scratch,
    )


def kernel(x, edge_index, batch, W1a, b1a, W2a, b2a, W1b, b1b, W2b, b2b,
           lin_w, lin_b):
    x = x.astype(jnp.float32)
    pad = E_PAD - E
    srcp = jnp.concatenate([edge_index[0], jnp.zeros((pad,), jnp.int32)])
    dstp = jnp.concatenate([edge_index[1], jnp.full((pad,), N, jnp.int32)])
    src2 = srcp.reshape(CH_TOT_PAD, CHUNK)
    dst2 = dstp.reshape(CH_TOT_PAD, CHUNK)
    batch3 = batch.reshape(GRID, 1, BN)
    b1a_ = b1a.reshape(1, D)
    b2a_ = b2a.reshape(1, D)
    b1b_ = b1b.reshape(1, D)
    b2b_ = b2b.reshape(1, D)
    lwT = jnp.broadcast_to(lin_w.reshape(D, 1), (D, D))
    lb_ = jnp.broadcast_to(lin_b.reshape(1, 1), (1, D))

    agg1 = _sc_aggregate(x, src2, dst2)
    h1 = _make_mlp(False)(x, agg1[0, :N], agg1[1, :N], W1a, b1a_, W2a, b2a_)
    agg2 = _sc_aggregate(h1, src2, dst2)
    h2, out_mat = _make_mlp(True)(h1, agg2[0, :N], agg2[1, :N], W1b, b1b_,
                                  W2b, b2b_, batch3, lwT, lb_)
    del h2
    return out_mat[:, 0]


# speedup vs baseline: 1.0178x; 1.0178x over previous
import functools

import jax
import jax.numpy as jnp
from jax import lax
from jax.experimental import pallas as pl
from jax.experimental.pallas import tpu as pltpu
from jax.experimental.pallas import tpu_sc as plsc

N = 10000
D = 128
E = 320000
G = 64

NC = 2
NS = 16

CHUNK = 128
CH_TOT = -(-E // CHUNK)
CPT = ((CH_TOT + NS - 1) // NS + 7) // 8 * 8
CH_TOT_PAD = CPT * NS
E_PAD = CH_TOT_PAD * CHUNK

N_PAD = 10240
ROWS_PER_TILE = N_PAD // NS

NBUF = 2
SS = 8
FAST_CORE = 0
CPT_FAST = 136
CPT_SLOW = 24


def _agg_body(x_hbm, src_hbm, dst_hbm, out_hbm, src_idx, dst_idx, bufs, acc,
              sems):
    cid = lax.axis_index("c")
    tid = lax.axis_index("s")

    rows = bufs[0]

    def zero_body(t, _):
        i = t // (D // 16)
        k = t % (D // 16)
        rows[i, pl.ds(k * 16, 16)] = jnp.zeros((16,), jnp.float32)
        return 0
    lax.fori_loop(0, CHUNK * (D // 16), zero_body, 0)
    base = tid * ROWS_PER_TILE
    for c in range(ROWS_PER_TILE // CHUNK):
        pltpu.sync_copy(rows, acc.at[pl.ds(base + c * CHUNK, CHUNK)])
    plsc.subcore_barrier()

    def pipeline(chunk_base, cpt):
        for st in range(cpt // SS):
            off = chunk_base + st * SS
            pltpu.sync_copy(src_hbm.at[pl.ds(off, SS)], src_idx)
            pltpu.sync_copy(dst_hbm.at[pl.ds(off, SS)], dst_idx)

            for b in range(NBUF):
                pltpu.async_copy(x_hbm.at[src_idx.at[b]], bufs[b], sems[b])

            def group_body(g, _):
                for b in range(NBUF):
                    j = g * NBUF + b
                    pltpu.make_async_copy(x_hbm.at[src_idx.at[0]], bufs[b],
                                          sems[b]).wait()
                    pltpu.sync_copy(bufs[b], acc.at[dst_idx.at[j]], add=True)
                    nj = j + NBUF

                    @pl.when(nj < SS)
                    def _():
                        pltpu.async_copy(x_hbm.at[src_idx.at[nj]], bufs[b],
                                         sems[b])
                return 0
            lax.fori_loop(0, SS // NBUF, group_body, 0)

    @pl.when(cid == FAST_CORE)
    def _():
        pipeline(tid * CPT_FAST, CPT_FAST)

    @pl.when(cid == 1 - FAST_CORE)
    def _():
        pipeline(NS * CPT_FAST + tid * CPT_SLOW, CPT_SLOW)

    plsc.subcore_barrier()

    pltpu.sync_copy(acc.at[pl.ds(base, ROWS_PER_TILE)],
                    out_hbm.at[cid].at[pl.ds(base, ROWS_PER_TILE)])


@functools.partial(
    pl.kernel,
    out_type=jax.ShapeDtypeStruct((NC, N_PAD, D), jnp.float32),
    mesh=plsc.VectorSubcoreMesh(core_axis_name="c", subcore_axis_name="s"),
    scratch_types=[
        pltpu.VMEM((SS, CHUNK), jnp.int32),
        pltpu.VMEM((SS, CHUNK), jnp.int32),
        pltpu.VMEM((CHUNK, D), jnp.float32),
        pltpu.VMEM((CHUNK, D), jnp.float32),
        pltpu.VMEM_SHARED((N_PAD, D), jnp.float32),
        pltpu.SemaphoreType.DMA,
        pltpu.SemaphoreType.DMA,
    ],
)
def _sc_aggregate(x_hbm, src_hbm, dst_hbm, out_hbm, src_idx, dst_idx,
                  b0, b1, acc, s0, s1):
    _agg_body(x_hbm, src_hbm, dst_hbm, out_hbm, src_idx, dst_idx,
              [b0, b1], acc, [s0, s1])


BN = 2000
GRID = N // BN


def _mlp_body(do_pool, x_ref, a0_ref, a1_ref, w1_ref, b1_ref, w2_ref, b2_ref,
              *rest):
    if do_pool:
        (batch_ref, lw_ref, lb_ref, h_ref, out_ref, pooled) = rest
    else:
        (h_ref,) = rest
    h = x_ref[...] + a0_ref[...] + a1_ref[...]
    h = jnp.maximum(
        lax.dot_general(h, w1_ref[...], (((1,), (0,)), ((), ())),
                        preferred_element_type=jnp.float32) + b1_ref[...], 0.0)
    h = lax.dot_general(h, w2_ref[...], (((1,), (0,)), ((), ())),
                        preferred_element_type=jnp.float32) + b2_ref[...]
    h = jnp.maximum(h, 0.0)
    h_ref[...] = h
    if do_pool:
        i = pl.program_id(0)

        @pl.when(i == 0)
        def _():
            pooled[...] = jnp.zeros((G, D), jnp.float32)

        seg = batch_ref[0]
        oh = (lax.broadcasted_iota(jnp.int32, (G, BN), 0) == seg
              ).astype(jnp.float32)
        pooled[...] += lax.dot_general(oh, h, (((1,), (0,)), ((), ())),
                                       preferred_element_type=jnp.float32)

        @pl.when(i == GRID - 1)
        def _():
            z = lax.dot_general(pooled[...], lw_ref[...],
                                (((1,), (0,)), ((), ())),
                                preferred_element_type=jnp.float32)
            z = z + lb_ref[0, 0]
            out_ref[...] = 1.0 / (1.0 + jnp.exp(-z))


def _make_mlp(do_pool):
    in_specs = [
        pl.BlockSpec((BN, D), lambda i: (i, 0)),
        pl.BlockSpec((BN, D), lambda i: (i, 0)),
        pl.BlockSpec((BN, D), lambda i: (i, 0)),
        pl.BlockSpec((D, D), lambda i: (0, 0)),
        pl.BlockSpec((1, D), lambda i: (0, 0)),
        pl.BlockSpec((D, D), lambda i: (0, 0)),
        pl.BlockSpec((1, D), lambda i: (0, 0)),
    ]
    out_specs = pl.BlockSpec((BN, D), lambda i: (i, 0))
    out_shape = jax.ShapeDtypeStruct((N, D), jnp.float32)
    scratch = []
    if do_pool:
        in_specs += [
            pl.BlockSpec((1, 1, BN), lambda i: (i, 0, 0)),
            pl.BlockSpec((D, D), lambda i: (0, 0)),
            pl.BlockSpec((1, D), lambda i: (0, 0)),
        ]
        out_specs = [out_specs, pl.BlockSpec((G, D), lambda i: (0, 0))]
        out_shape = [out_shape, jax.ShapeDtypeStruct((G, D), jnp.float32)]
        scratch = [pltpu.VMEM((G, D), jnp.float32)]
    return pl.pallas_call(
        functools.partial(_mlp_body, do_pool),
        grid=(GRID,),
        in_specs=in_specs,
        out_specs=out_specs,
        out_shape=out_shape,
        scratch_shapes=scratch,
    )


def kernel(x, edge_index, batch, W1a, b1a, W2a, b2a, W1b, b1b, W2b, b2b,
           lin_w, lin_b):
    x = x.astype(jnp.float32)
    pad = E_PAD - E
    srcp = jnp.concatenate([edge_index[0], jnp.zeros((pad,), jnp.int32)])
    dstp = jnp.concatenate([edge_index[1], jnp.full((pad,), N, jnp.int32)])
    src2 = srcp.reshape(CH_TOT_PAD, CHUNK)
    dst2 = dstp.reshape(CH_TOT_PAD, CHUNK)
    batch3 = batch.reshape(GRID, 1, BN)
    b1a_ = b1a.reshape(1, D)
    b2a_ = b2a.reshape(1, D)
    b1b_ = b1b.reshape(1, D)
    b2b_ = b2b.reshape(1, D)
    lwT = jnp.broadcast_to(lin_w.reshape(D, 1), (D, D))
    lb_ = jnp.broadcast_to(lin_b.reshape(1, 1), (1, D))

    agg1 = _sc_aggregate(x, src2, dst2)
    h1 = _make_mlp(False)(x, agg1[0, :N], agg1[1, :N], W1a, b1a_, W2a, b2a_)
    agg2 = _sc_aggregate(h1, src2, dst2)
    h2, out_mat = _make_mlp(True)(h1, agg2[0, :N], agg2[1, :N], W1b, b1b_,
                                  W2b, b2b_, batch3, lwT, lb_)
    del h2
    return out_mat[:, 0]


# speedup vs baseline: 1.0781x; 1.0593x over previous
import functools

import jax
import jax.numpy as jnp
from jax import lax
from jax.experimental import pallas as pl
from jax.experimental.pallas import tpu as pltpu
from jax.experimental.pallas import tpu_sc as plsc

N = 10000
D = 128
E = 320000
G = 64

NC = 2
NS = 16

CHUNK = 128
CH_TOT = -(-E // CHUNK)
CPT = ((CH_TOT + NS - 1) // NS + 7) // 8 * 8
CH_TOT_PAD = CPT * NS
E_PAD = CH_TOT_PAD * CHUNK

N_PAD = 10240
ROWS_PER_TILE = N_PAD // NS

NBUF = 2
SS = 16
FAST_CORE = 0
CPT_FAST = 144
CPT_SLOW = 16


def _agg_body(x_hbm, src_hbm, dst_hbm, out_hbm, src_idx, dst_idx, bufs, acc,
              sems):
    cid = lax.axis_index("c")
    tid = lax.axis_index("s")

    rows = bufs[0]

    def zero_body(t, _):
        i = t // (D // 16)
        k = t % (D // 16)
        rows[i, pl.ds(k * 16, 16)] = jnp.zeros((16,), jnp.float32)
        return 0
    lax.fori_loop(0, CHUNK * (D // 16), zero_body, 0)
    base = tid * ROWS_PER_TILE
    for c in range(ROWS_PER_TILE // CHUNK):
        pltpu.sync_copy(rows, acc.at[pl.ds(base + c * CHUNK, CHUNK)])
    plsc.subcore_barrier()

    def pipeline(chunk_base, cpt):
        for st in range(cpt // SS):
            off = chunk_base + st * SS
            pltpu.sync_copy(src_hbm.at[pl.ds(off, SS)], src_idx)
            pltpu.sync_copy(dst_hbm.at[pl.ds(off, SS)], dst_idx)

            for b in range(NBUF):
                pltpu.async_copy(x_hbm.at[src_idx.at[b]], bufs[b], sems[b])

            def group_body(g, _):
                for b in range(NBUF):
                    j = g * NBUF + b
                    pltpu.make_async_copy(x_hbm.at[src_idx.at[0]], bufs[b],
                                          sems[b]).wait()
                    pltpu.sync_copy(bufs[b], acc.at[dst_idx.at[j]], add=True)
                    nj = j + NBUF

                    @pl.when(nj < SS)
                    def _():
                        pltpu.async_copy(x_hbm.at[src_idx.at[nj]], bufs[b],
                                         sems[b])
                return 0
            lax.fori_loop(0, SS // NBUF, group_body, 0)

    @pl.when(cid == FAST_CORE)
    def _():
        pipeline(tid * CPT_FAST, CPT_FAST)

    @pl.when(cid == 1 - FAST_CORE)
    def _():
        pipeline(NS * CPT_FAST + tid * CPT_SLOW, CPT_SLOW)

    plsc.subcore_barrier()

    pltpu.sync_copy(acc.at[pl.ds(base, ROWS_PER_TILE)],
                    out_hbm.at[cid].at[pl.ds(base, ROWS_PER_TILE)])


@functools.partial(
    pl.kernel,
    out_type=jax.ShapeDtypeStruct((NC, N_PAD, D), jnp.float32),
    mesh=plsc.VectorSubcoreMesh(core_axis_name="c", subcore_axis_name="s"),
    scratch_types=[
        pltpu.VMEM((SS, CHUNK), jnp.int32),
        pltpu.VMEM((SS, CHUNK), jnp.int32),
        pltpu.VMEM((CHUNK, D), jnp.float32),
        pltpu.VMEM((CHUNK, D), jnp.float32),
        pltpu.VMEM_SHARED((N_PAD, D), jnp.float32),
        pltpu.SemaphoreType.DMA,
        pltpu.SemaphoreType.DMA,
    ],
)
def _sc_aggregate(x_hbm, src_hbm, dst_hbm, out_hbm, src_idx, dst_idx,
                  b0, b1, acc, s0, s1):
    _agg_body(x_hbm, src_hbm, dst_hbm, out_hbm, src_idx, dst_idx,
              [b0, b1], acc, [s0, s1])


BN = 2000
GRID = N // BN


def _mlp_body(do_pool, x_ref, a0_ref, a1_ref, w1_ref, b1_ref, w2_ref, b2_ref,
              *rest):
    if do_pool:
        (batch_ref, lw_ref, lb_ref, h_ref, out_ref, pooled) = rest
    else:
        (h_ref,) = rest
    h = x_ref[...] + a0_ref[...] + a1_ref[...]
    h = jnp.maximum(
        lax.dot_general(h, w1_ref[...], (((1,), (0,)), ((), ())),
                        preferred_element_type=jnp.float32) + b1_ref[...], 0.0)
    h = lax.dot_general(h, w2_ref[...], (((1,), (0,)), ((), ())),
                        preferred_element_type=jnp.float32) + b2_ref[...]
    h = jnp.maximum(h, 0.0)
    h_ref[...] = h
    if do_pool:
        i = pl.program_id(0)

        @pl.when(i == 0)
        def _():
            pooled[...] = jnp.zeros((G, D), jnp.float32)

        seg = batch_ref[0]
        oh = (lax.broadcasted_iota(jnp.int32, (G, BN), 0) == seg
              ).astype(jnp.float32)
        pooled[...] += lax.dot_general(oh, h, (((1,), (0,)), ((), ())),
                                       preferred_element_type=jnp.float32)

        @pl.when(i == GRID - 1)
        def _():
            z = lax.dot_general(pooled[...], lw_ref[...],
                                (((1,), (0,)), ((), ())),
                                preferred_element_type=jnp.float32)
            z = z + lb_ref[0, 0]
            out_ref[...] = 1.0 / (1.0 + jnp.exp(-z))


def _make_mlp(do_pool):
    in_specs = [
        pl.BlockSpec((BN, D), lambda i: (i, 0)),
        pl.BlockSpec((BN, D), lambda i: (i, 0)),
        pl.BlockSpec((BN, D), lambda i: (i, 0)),
        pl.BlockSpec((D, D), lambda i: (0, 0)),
        pl.BlockSpec((1, D), lambda i: (0, 0)),
        pl.BlockSpec((D, D), lambda i: (0, 0)),
        pl.BlockSpec((1, D), lambda i: (0, 0)),
    ]
    out_specs = pl.BlockSpec((BN, D), lambda i: (i, 0))
    out_shape = jax.ShapeDtypeStruct((N, D), jnp.float32)
    scratch = []
    if do_pool:
        in_specs += [
            pl.BlockSpec((1, 1, BN), lambda i: (i, 0, 0)),
            pl.BlockSpec((D, D), lambda i: (0, 0)),
            pl.BlockSpec((1, D), lambda i: (0, 0)),
        ]
        out_specs = [out_specs, pl.BlockSpec((G, D), lambda i: (0, 0))]
        out_shape = [out_shape, jax.ShapeDtypeStruct((G, D), jnp.float32)]
        scratch = [pltpu.VMEM((G, D), jnp.float32)]
    return pl.pallas_call(
        functools.partial(_mlp_body, do_pool),
        grid=(GRID,),
        in_specs=in_specs,
        out_specs=out_specs,
        out_shape=out_shape,
        scratch_shapes=scratch,
    )


def kernel(x, edge_index, batch, W1a, b1a, W2a, b2a, W1b, b1b, W2b, b2b,
           lin_w, lin_b):
    x = x.astype(jnp.float32)
    pad = E_PAD - E
    srcp = jnp.concatenate([edge_index[0], jnp.zeros((pad,), jnp.int32)])
    dstp = jnp.concatenate([edge_index[1], jnp.full((pad,), N, jnp.int32)])
    src2 = srcp.reshape(CH_TOT_PAD, CHUNK)
    dst2 = dstp.reshape(CH_TOT_PAD, CHUNK)
    batch3 = batch.reshape(GRID, 1, BN)
    b1a_ = b1a.reshape(1, D)
    b2a_ = b2a.reshape(1, D)
    b1b_ = b1b.reshape(1, D)
    b2b_ = b2b.reshape(1, D)
    lwT = jnp.broadcast_to(lin_w.reshape(D, 1), (D, D))
    lb_ = jnp.broadcast_to(lin_b.reshape(1, 1), (1, D))

    agg1 = _sc_aggregate(x, src2, dst2)
    h1 = _make_mlp(False)(x, agg1[0, :N], agg1[1, :N], W1a, b1a_, W2a, b2a_)
    agg2 = _sc_aggregate(h1, src2, dst2)
    h2, out_mat = _make_mlp(True)(h1, agg2[0, :N], agg2[1, :N], W1b, b1b_,
                                  W2b, b2b_, batch3, lwT, lb_)
    del h2
    return out_mat[:, 0]
